# baseline (speedup 1.0000x reference)
import jax
import jax.numpy as jnp
from jax.experimental import pallas as pl


def _mm_kernel(x_ref, w_ref, o_ref):
    o_ref[...] = jnp.dot(x_ref[...], w_ref[...], preferred_element_type=jnp.float32)


def _mm(x, w):
    n, k = x.shape
    blk = 512
    npad = (n + blk - 1) // blk * blk
    xp = jnp.pad(x, ((0, npad - n), (0, 0)))
    out = pl.pallas_call(
        _mm_kernel,
        grid=(npad // blk,),
        in_specs=[
            pl.BlockSpec((blk, k), lambda i: (i, 0)),
            pl.BlockSpec((k, w.shape[1]), lambda i: (0, 0)),
        ],
        out_specs=pl.BlockSpec((blk, w.shape[1]), lambda i: (i, 0)),
        out_shape=jax.ShapeDtypeStruct((npad, w.shape[1]), jnp.float32),
    )(xp, w)
    return out[:n]


def kernel(q_sub, q_rel, hidden, edges, nodes, old_nodes_new_idx, batchsize,
           rela_embed, Ws_attn, Wr_attn, Wqr_attn, bqr, w_alpha, b_alpha, W_h,
           attn_fc_w, attn_fc_b, W_node_w, W_node_b):
    sub = edges[:, 4]
    rel = edges[:, 2]
    obj = edges[:, 5]
    r_idx = edges[:, 0]
    n_node = nodes.shape[0]

    A_s = _mm(hidden, Ws_attn)
    A_r = _mm(rela_embed, Wr_attn)
    A_q = _mm(rela_embed, Wqr_attn)
    QA = jnp.take(A_q, q_rel, axis=0)

    t = (jnp.take(A_s, sub, axis=0) + jnp.take(A_r, rel, axis=0)
         + jnp.take(QA, r_idx, axis=0) + bqr)
    attn_h = jax.nn.relu(t)
    alpha = jax.nn.sigmoid(attn_h @ w_alpha + b_alpha)
    message = alpha * (jnp.take(hidden, sub, axis=0) - jnp.take(rela_embed, rel, axis=0))
    agg = jax.ops.segment_max(message, obj, num_segments=n_node)
    agg = jnp.where(jnp.isfinite(agg), agg, 0.0)

    hidden_new = _mm(agg, W_h)
    s1 = hidden_new @ attn_fc_w[:128]
    s2 = hidden_new @ attn_fc_w[128:]
    scores = jax.nn.leaky_relu(
        (jnp.take(s1, sub, axis=0) + jnp.take(s2, obj, axis=0) + attn_fc_b).squeeze(-1),
        negative_slope=0.2)
    weights = jax.nn.softmax(scores)
    H2 = _mm(hidden_new, W_node_w) + W_node_b
    vals = weights[:, None] * jnp.take(H2, obj, axis=0)
    out = jax.ops.segment_sum(vals, sub, num_segments=n_node)
    return out
